# baseline (speedup 1.0000x reference)
import functools

import jax
import jax.numpy as jnp
import numpy as np
from jax.experimental import pallas as pl
from jax.experimental.pallas import tpu as pltpu

ALPHA_ = 0.99
EPS_ = 1e-6
EMA_INIT_ = 1e-4
L_ = 512
S_ = 8


def _ema_norm_kernel(x_ref, a_ref, pow_ref, gamma_ref, beta_ref, o_ref,
                     carry_ref, *, L, S):
    chunk = pl.program_id(1)

    @pl.when(chunk == 0)
    def _():
        carry_ref[...] = jnp.full_like(carry_ref, EMA_INIT_)

    amat = a_ref[...]
    powv = pow_ref[...]
    ln_a = float(np.log(ALPHA_))
    gamma = gamma_ref[...]
    beta = beta_ref[...]

    carry = carry_ref[...]
    for k in range(S):
        x = x_ref[0, k * L:(k + 1) * L, :]
        acc = jnp.dot(amat, x * x, preferred_element_type=jnp.float32)
        ema = acc + powv * carry
        carry = ema[L - 1:L, :]
        s = jnp.exp((chunk.astype(jnp.float32) * S + k) * (L * ln_a))
        v = ema / ((1.0 + EPS_) - s * powv) + EPS_
        gn = v * jax.lax.rsqrt(v)
        m = jnp.mean(gn, axis=-1, keepdims=True)
        n = gn / (m + EPS_)
        o_ref[0, k * L:(k + 1) * L, :] = gamma * (x * n) + beta + x
    carry_ref[...] = carry


@jax.jit
def kernel(x, gamma, beta):
    B, T, C = x.shape
    L = L_
    S = S_
    num_chunks = T // (S * L)

    i = np.arange(L)
    amat = np.where(i[:, None] >= i[None, :],
                    (1.0 - ALPHA_) * ALPHA_ ** (i[:, None] - i[None, :]), 0.0)
    amat = jnp.asarray(amat, dtype=jnp.float32)
    powv = jnp.asarray(
        np.broadcast_to((ALPHA_ ** (i + 1))[:, None], (L, C)).copy(),
        dtype=jnp.float32)

    grid = (B, num_chunks)
    out = pl.pallas_call(
        functools.partial(_ema_norm_kernel, L=L, S=S),
        grid=grid,
        in_specs=[
            pl.BlockSpec((1, S * L, C), lambda b, t: (b, t, 0)),
            pl.BlockSpec((L, L), lambda b, t: (0, 0)),
            pl.BlockSpec((L, C), lambda b, t: (0, 0)),
            pl.BlockSpec((1, C), lambda b, t: (0, 0)),
            pl.BlockSpec((1, C), lambda b, t: (0, 0)),
        ],
        out_specs=pl.BlockSpec((1, S * L, C), lambda b, t: (b, t, 0)),
        out_shape=jax.ShapeDtypeStruct((B, T, C), x.dtype),
        scratch_shapes=[pltpu.VMEM((1, C), jnp.float32)],
        compiler_params=pltpu.CompilerParams(
            dimension_semantics=("parallel", "arbitrary"),
        ),
    )(x, amat, powv, gamma, beta)
    return out


# speedup vs baseline: 1.0621x; 1.0621x over previous
import functools

import jax
import jax.numpy as jnp
import numpy as np
from jax.experimental import pallas as pl
from jax.experimental.pallas import tpu as pltpu

ALPHA_ = 0.99
EPS_ = 1e-6
EMA_INIT_ = 1e-4
L_ = 512
G_ = 8


def _ema_norm_kernel(x_ref, a_ref, pow_ref, gamma_ref, beta_ref, o_ref,
                     carry_ref, *, L, G):
    chunk = pl.program_id(1)

    @pl.when(chunk == 0)
    def _():
        carry_ref[...] = jnp.full_like(carry_ref, EMA_INIT_)

    amat = a_ref[...]
    powv = pow_ref[...]
    ln_a = float(np.log(ALPHA_))
    s = jnp.exp(chunk.astype(jnp.float32) * (L * ln_a))
    denom = (1.0 + EPS_) - s * powv
    gamma = gamma_ref[...]
    beta = beta_ref[...]

    for g in range(G):
        x = x_ref[g]
        acc = jnp.dot(amat, x * x, preferred_element_type=jnp.float32)
        ema = acc + powv * carry_ref[g:g + 1]
        carry_ref[g:g + 1] = ema[L - 1:L, :]
        v = ema / denom + EPS_
        gn = v * jax.lax.rsqrt(v)
        m = jnp.mean(gn, axis=-1, keepdims=True)
        n = gn / (m + EPS_)
        o_ref[g] = gamma * (x * n) + beta + x


@jax.jit
def kernel(x, gamma, beta):
    B, T, C = x.shape
    L = L_
    G = G_
    num_chunks = T // L

    i = np.arange(L)
    amat = np.where(i[:, None] >= i[None, :],
                    (1.0 - ALPHA_) * ALPHA_ ** (i[:, None] - i[None, :]), 0.0)
    amat = jnp.asarray(amat, dtype=jnp.float32)
    powv = jnp.asarray(
        np.broadcast_to((ALPHA_ ** (i + 1))[:, None], (L, C)).copy(),
        dtype=jnp.float32)

    grid = (B // G, num_chunks)
    out = pl.pallas_call(
        functools.partial(_ema_norm_kernel, L=L, G=G),
        grid=grid,
        in_specs=[
            pl.BlockSpec((G, L, C), lambda b, t: (b, t, 0)),
            pl.BlockSpec((L, L), lambda b, t: (0, 0)),
            pl.BlockSpec((L, C), lambda b, t: (0, 0)),
            pl.BlockSpec((1, C), lambda b, t: (0, 0)),
            pl.BlockSpec((1, C), lambda b, t: (0, 0)),
        ],
        out_specs=pl.BlockSpec((G, L, C), lambda b, t: (b, t, 0)),
        out_shape=jax.ShapeDtypeStruct((B, T, C), x.dtype),
        scratch_shapes=[pltpu.VMEM((G, C), jnp.float32)],
        compiler_params=pltpu.CompilerParams(
            dimension_semantics=("parallel", "arbitrary"),
        ),
    )(x, amat, powv, gamma, beta)
    return out
